# baseline (speedup 1.0000x reference)
import functools
import math

import jax
import jax.numpy as jnp
from jax import lax
from jax.experimental import pallas as pl
from jax.experimental.pallas import tpu as pltpu
from jax.experimental.pallas import tpu_sc as plsc

D_MODEL = 768
SCALE = math.sqrt(D_MODEL)
NC, NS, LANES = 2, 16, 16
NW = NC * NS
CHUNK = 32
NBUF = 5
LAG = 2


def _scale_chunk(buf):
    def row_body(r, carry):
        for c in range(D_MODEL // LANES):
            sl = pl.ds(c * LANES, LANES)
            buf[r, sl] = buf[r, sl] * SCALE
        return carry

    lax.fori_loop(0, CHUNK, row_body, 0)


def _emb_body(nchunks, b_per_w, x_hbm, tab_hbm, out_hbm, idx_v, rows_v, *sems):
    gs, ss = sems[:NBUF], sems[NBUF:]
    wid = lax.axis_index("s") * NC + lax.axis_index("c")
    base = wid * b_per_w
    pltpu.sync_copy(x_hbm.at[wid], idx_v)

    def start_gather(j, b):
        pltpu.async_copy(tab_hbm.at[idx_v.at[j]], rows_v.at[b], gs[b])

    def wait_gather(b):
        pltpu.make_async_copy(tab_hbm.at[idx_v.at[0]], rows_v.at[b], gs[b]).wait()

    def start_store(j, b):
        dst = out_hbm.at[pl.ds(base + j * CHUNK, CHUNK)]
        pltpu.async_copy(rows_v.at[b], dst, ss[b])

    def wait_store(b):
        dst = out_hbm.at[pl.ds(base, CHUNK)]
        pltpu.make_async_copy(rows_v.at[b], dst, ss[b]).wait()

    for b in range(NBUF):
        start_gather(b, b)

    for k in range(nchunks):
        b = k % NBUF
        wait_gather(b)
        _scale_chunk(rows_v.at[b])
        start_store(k, b)
        kd = k - LAG
        if kd >= 0 and kd + NBUF < nchunks:
            wait_store(kd % NBUF)
            start_gather(kd + NBUF, kd % NBUF)

    for k in range(nchunks - NBUF, nchunks):
        wait_store(k % NBUF)


def kernel(x, embedding_weight):
    orig_shape = x.shape
    b_total = x.size
    b_per_w = b_total // NW
    nchunks = b_per_w // CHUNK
    x_resh = x.reshape(NW, nchunks, CHUNK).astype(jnp.int32)

    mesh = plsc.VectorSubcoreMesh(core_axis_name="c", subcore_axis_name="s")
    emb = pl.kernel(
        functools.partial(_emb_body, nchunks, b_per_w),
        out_type=jax.ShapeDtypeStruct((b_total, D_MODEL), jnp.float32),
        mesh=mesh,
        scratch_types=[
            pltpu.VMEM((nchunks, CHUNK), jnp.int32),
            pltpu.VMEM((NBUF, CHUNK, D_MODEL), jnp.float32),
        ] + [pltpu.SemaphoreType.DMA] * (2 * NBUF),
    )
    out = emb(x_resh, embedding_weight)
    return out.reshape(orig_shape + (D_MODEL,))


# speedup vs baseline: 1.0384x; 1.0384x over previous
import functools
import math

import jax
import jax.numpy as jnp
from jax import lax
from jax.experimental import pallas as pl
from jax.experimental.pallas import tpu as pltpu
from jax.experimental.pallas import tpu_sc as plsc

D_MODEL = 768
SCALE = math.sqrt(D_MODEL)
NC, NS, LANES = 2, 16, 16
NW = NC * NS
CHUNK = 64
NBUF = 2
SPLITS = 2
SUB = CHUNK // SPLITS


def _scale_rows(buf, start, nrows):
    def row_body(r, carry):
        for c in range(D_MODEL // LANES):
            sl = pl.ds(c * LANES, LANES)
            buf[r, sl] = buf[r, sl] * SCALE
        return carry

    lax.fori_loop(start, start + nrows, row_body, 0)


def _emb_body(nchunks, b_per_w, x_hbm, tab_hbm, out_hbm, idx_v, rows_v, *sems):
    gs, ss = sems[:SPLITS * NBUF], sems[SPLITS * NBUF:]
    wid = lax.axis_index("s") * NC + lax.axis_index("c")
    base = wid * b_per_w
    pltpu.sync_copy(x_hbm.at[wid], idx_v)

    def start_gather_half(j, b, h):
        src = tab_hbm.at[idx_v.at[SPLITS * j + h]]
        dst = rows_v.at[b].at[pl.ds(h * SUB, SUB)]
        pltpu.async_copy(src, dst, gs[SPLITS * b + h])

    def wait_gather(b, h):
        dst = rows_v.at[b].at[pl.ds(h * SUB, SUB)]
        pltpu.make_async_copy(tab_hbm.at[idx_v.at[0]], dst, gs[SPLITS * b + h]).wait()

    def start_store_half(j, b, h):
        src = rows_v.at[b].at[pl.ds(h * SUB, SUB)]
        dst = out_hbm.at[pl.ds(base + j * CHUNK + h * SUB, SUB)]
        pltpu.async_copy(src, dst, ss[SPLITS * b + h])

    def wait_store_half(b, h):
        dst = out_hbm.at[pl.ds(base, SUB)]
        src = rows_v.at[b].at[pl.ds(h * SUB, SUB)]
        pltpu.make_async_copy(src, dst, ss[SPLITS * b + h]).wait()

    def piece(j, b, h):
        wait_gather(b, h)
        _scale_rows(rows_v.at[b], h * SUB, SUB)
        start_store_half(j, b, h)

    for b in range(NBUF):
        for h in range(SPLITS):
            start_gather_half(b, b, h)

    ngroups = nchunks // NBUF

    def group_body(g, carry):
        j0 = g * NBUF
        jn = (g + 1) * NBUF
        for h in range(SPLITS):
            piece(j0, 0, h)
        for h in range(SPLITS):
            piece(j0 + 1, 1, h)
            wait_store_half(0, h)
            start_gather_half(jn, 0, h)
        for h in range(SPLITS):
            wait_store_half(1, h)
            start_gather_half(jn + 1, 1, h)
        return carry

    lax.fori_loop(0, ngroups - 1, group_body, 0)

    g = ngroups - 1
    for b in range(NBUF):
        for h in range(SPLITS):
            piece(g * NBUF + b, b, h)
    for b in range(NBUF):
        for h in range(SPLITS):
            wait_store_half(b, h)


def kernel(x, embedding_weight):
    orig_shape = x.shape
    b_total = x.size
    b_per_w = b_total // NW
    nchunks = b_per_w // CHUNK
    x_resh = x.reshape(NW, SPLITS * nchunks, SUB).astype(jnp.int32)

    mesh = plsc.VectorSubcoreMesh(core_axis_name="c", subcore_axis_name="s")
    emb = pl.kernel(
        functools.partial(_emb_body, nchunks, b_per_w),
        out_type=jax.ShapeDtypeStruct((b_total, D_MODEL), jnp.float32),
        mesh=mesh,
        scratch_types=[
            pltpu.VMEM((SPLITS * nchunks, SUB), jnp.int32),
            pltpu.VMEM((NBUF, CHUNK, D_MODEL), jnp.float32),
        ] + [pltpu.SemaphoreType.DMA] * (2 * SPLITS * NBUF),
    )
    out = emb(x_resh, embedding_weight)
    return out.reshape(orig_shape + (D_MODEL,))


# speedup vs baseline: 1.0728x; 1.0332x over previous
import functools
import math

import jax
import jax.numpy as jnp
from jax import lax
from jax.experimental import pallas as pl
from jax.experimental.pallas import tpu as pltpu
from jax.experimental.pallas import tpu_sc as plsc

D_MODEL = 768
SCALE = math.sqrt(D_MODEL)
NC, NS, LANES = 2, 16, 16
NW = NC * NS
CHUNK = 64
NBUF = 2
SPLITS = 2
SUB = CHUNK // SPLITS


def _scale_rows(buf, start, nrows):
    def row_body(r, carry):
        for c in range(D_MODEL // LANES):
            sl = pl.ds(c * LANES, LANES)
            buf[r, sl] = buf[r, sl] * SCALE
        return carry

    lax.fori_loop(start, start + nrows, row_body, 0)


def _emb_body(nchunks, b_per_w, x_hbm, tab_hbm, out_hbm, idx_v, rows_v, *sems):
    gs, ss = sems[:SPLITS * NBUF], sems[SPLITS * NBUF:]
    wid = lax.axis_index("s") * NC + lax.axis_index("c")
    base = wid * b_per_w
    pltpu.sync_copy(x_hbm.at[wid], idx_v)

    def start_gather_half(j, b, h):
        src = tab_hbm.at[idx_v.at[SPLITS * j + h]]
        dst = rows_v.at[b].at[pl.ds(h * SUB, SUB)]
        pltpu.async_copy(src, dst, gs[SPLITS * b + h])

    def wait_gather(b, h):
        dst = rows_v.at[b].at[pl.ds(h * SUB, SUB)]
        pltpu.make_async_copy(tab_hbm.at[idx_v.at[0]], dst, gs[SPLITS * b + h]).wait()

    def start_store_half(j, b, h):
        src = rows_v.at[b].at[pl.ds(h * SUB, SUB)]
        dst = out_hbm.at[pl.ds(base + j * CHUNK + h * SUB, SUB)]
        pltpu.async_copy(src, dst, ss[SPLITS * b + h])

    def wait_store_half(b, h):
        dst = out_hbm.at[pl.ds(base, SUB)]
        src = rows_v.at[b].at[pl.ds(h * SUB, SUB)]
        pltpu.make_async_copy(src, dst, ss[SPLITS * b + h]).wait()

    def piece(j, b, h):
        wait_gather(b, h)
        _scale_rows(rows_v.at[b], h * SUB, SUB)
        start_store_half(j, b, h)

    for b in range(NBUF):
        for h in range(SPLITS):
            start_gather_half(b, b, h)

    ngroups = nchunks // NBUF

    def group_body(g, carry):
        for b in range(NBUF):
            for h in range(SPLITS):
                piece(g * NBUF + b, b, h)
            for h in range(SPLITS):
                wait_store_half(b, h)
                start_gather_half((g + 1) * NBUF + b, b, h)
        return carry

    lax.fori_loop(0, ngroups - 1, group_body, 0)

    g = ngroups - 1
    for b in range(NBUF):
        for h in range(SPLITS):
            piece(g * NBUF + b, b, h)
    for b in range(NBUF):
        for h in range(SPLITS):
            wait_store_half(b, h)


def kernel(x, embedding_weight):
    orig_shape = x.shape
    b_total = x.size
    b_per_w = b_total // NW
    nchunks = b_per_w // CHUNK
    x_resh = x.reshape(NW, SPLITS * nchunks, SUB).astype(jnp.int32)

    mesh = plsc.VectorSubcoreMesh(core_axis_name="c", subcore_axis_name="s")
    emb = pl.kernel(
        functools.partial(_emb_body, nchunks, b_per_w),
        out_type=jax.ShapeDtypeStruct((b_total, D_MODEL), jnp.float32),
        mesh=mesh,
        scratch_types=[
            pltpu.VMEM((SPLITS * nchunks, SUB), jnp.int32),
            pltpu.VMEM((NBUF, CHUNK, D_MODEL), jnp.float32),
        ] + [pltpu.SemaphoreType.DMA] * (2 * SPLITS * NBUF),
    )
    out = emb(x_resh, embedding_weight)
    return out.reshape(orig_shape + (D_MODEL,))
